# speedup vs baseline: 5.1834x; 5.1834x over previous
import functools

import jax
import jax.numpy as jnp
from jax import lax
from jax.experimental import pallas as pl
from jax.experimental.pallas import tpu as pltpu
from jax.experimental.pallas import tpu_sc as plsc

_NC = 2
_NS = 16
_NW = _NC * _NS

_CH = 128


def _make_sc_gather(tot, emb):
    per_w = tot // _NW
    n_chunks = per_w // _CH
    mesh = plsc.VectorSubcoreMesh(core_axis_name="c", subcore_axis_name="s")

    @functools.partial(
        pl.kernel,
        mesh=mesh,
        out_type=jax.ShapeDtypeStruct((tot, emb), jnp.float32),
        scratch_types=[
            pltpu.VMEM((_CH,), jnp.int32),
            pltpu.VMEM((_CH, emb), jnp.float32),
            pltpu.SemaphoreType.DMA,
        ],
    )
    def sc_gather(idx_hbm, table_hbm, out_hbm, idx_v, rows_v, sem):
        wid = lax.axis_index("s") * _NC + lax.axis_index("c")
        base = wid * per_w

        def body(g, carry):
            off = base + g * _CH
            pltpu.sync_copy(idx_hbm.at[pl.ds(off, _CH)], idx_v)
            pltpu.async_copy(table_hbm.at[idx_v], rows_v, sem).wait()
            pltpu.sync_copy(rows_v, out_hbm.at[pl.ds(off, _CH)])
            return carry

        lax.fori_loop(0, n_chunks, body, 0)

    return sc_gather


def kernel(inputs, table):
    b, l = inputs.shape
    vocab, emb = table.shape
    tot = b * l
    idx_flat = inputs.reshape(tot).astype(jnp.int32)
    out = _make_sc_gather(tot, emb)(idx_flat, table)
    return out.reshape(b, l, emb)


# speedup vs baseline: 6.3582x; 1.2267x over previous
import functools

import jax
import jax.numpy as jnp
from jax import lax
from jax.experimental import pallas as pl
from jax.experimental.pallas import tpu as pltpu
from jax.experimental.pallas import tpu_sc as plsc

_NC = 2
_NS = 16
_NW = _NC * _NS

_CH = 128


def _make_sc_gather(tot, emb):
    per_w = tot // _NW
    n_chunks = per_w // _CH
    mesh = plsc.VectorSubcoreMesh(core_axis_name="c", subcore_axis_name="s")

    @functools.partial(
        pl.kernel,
        mesh=mesh,
        out_type=jax.ShapeDtypeStruct((tot, emb), jnp.float32),
        scratch_types=[
            pltpu.VMEM((n_chunks, _CH), jnp.int32),
            pltpu.VMEM((_CH, emb), jnp.float32),
            pltpu.SemaphoreType.DMA,
        ],
    )
    def sc_gather(idx_hbm, table_hbm, out_hbm, idx_v, rows_v, sem):
        wid = lax.axis_index("s") * _NC + lax.axis_index("c")
        base = wid * per_w
        pltpu.sync_copy(idx_hbm.at[wid], idx_v)

        def body(g, carry):
            off = base + g * _CH
            pltpu.async_copy(table_hbm.at[idx_v.at[g]], rows_v, sem).wait()
            pltpu.sync_copy(rows_v, out_hbm.at[pl.ds(off, _CH)])
            return carry

        lax.fori_loop(0, n_chunks, body, 0)

    return sc_gather


def kernel(inputs, table):
    b, l = inputs.shape
    vocab, emb = table.shape
    tot = b * l
    per_w = tot // _NW
    idx3 = inputs.reshape(_NW, per_w // _CH, _CH).astype(jnp.int32)
    out = _make_sc_gather(tot, emb)(idx3, table)
    return out.reshape(b, l, emb)


# speedup vs baseline: 9.1697x; 1.4422x over previous
import functools

import jax
import jax.numpy as jnp
from jax import lax
from jax.experimental import pallas as pl
from jax.experimental.pallas import tpu as pltpu
from jax.experimental.pallas import tpu_sc as plsc

_NC = 2
_NS = 16
_NW = _NC * _NS

_CH = 128


_NB = 4


def _make_sc_gather(tot, emb):
    per_w = tot // _NW
    n_chunks = per_w // _CH
    n_groups = n_chunks // _NB
    mesh = plsc.VectorSubcoreMesh(core_axis_name="c", subcore_axis_name="s")

    scratch = (
        [pltpu.VMEM((n_chunks, _CH), jnp.int32)]
        + [pltpu.VMEM((_CH, emb), jnp.float32) for _ in range(_NB)]
        + [pltpu.SemaphoreType.DMA for _ in range(2 * _NB)]
    )

    @functools.partial(
        pl.kernel,
        mesh=mesh,
        out_type=jax.ShapeDtypeStruct((tot, emb), jnp.float32),
        scratch_types=scratch,
    )
    def sc_gather(idx_hbm, table_hbm, out_hbm, idx_v, *bufs):
        rows = bufs[:_NB]
        gsem = bufs[_NB:2 * _NB]
        wsem = bufs[2 * _NB:]
        wid = lax.axis_index("s") * _NC + lax.axis_index("c")
        base = wid * per_w
        pltpu.sync_copy(idx_hbm.at[wid], idx_v)

        def gather(c, b):
            return pltpu.async_copy(table_hbm.at[idx_v.at[c]], rows[b], gsem[b])

        def write(c, b):
            return pltpu.async_copy(
                rows[b], out_hbm.at[pl.ds(base + c * _CH, _CH)], wsem[b])

        def wait_gather(c, b):
            pltpu.make_async_copy(
                table_hbm.at[idx_v.at[c]], rows[b], gsem[b]).wait()

        def wait_write(c, b):
            pltpu.make_async_copy(
                rows[b], out_hbm.at[pl.ds(base + c * _CH, _CH)], wsem[b]).wait()

        for b in range(_NB):
            gather(b, b)
        for b in range(_NB):
            wait_gather(b, b)
            write(b, b)

        def body(t, carry):
            c0 = t * _NB
            for b in range(_NB):
                wait_write(c0 + b - _NB, b)
                gather(c0 + b, b)
            for b in range(_NB):
                wait_gather(c0 + b, b)
                write(c0 + b, b)
            return carry

        lax.fori_loop(1, n_groups, body, 0)

        for b in range(_NB):
            wait_write((n_groups - 1) * _NB + b, b)

    return sc_gather


def kernel(inputs, table):
    b, l = inputs.shape
    vocab, emb = table.shape
    tot = b * l
    per_w = tot // _NW
    idx3 = inputs.reshape(_NW, per_w // _CH, _CH).astype(jnp.int32)
    out = _make_sc_gather(tot, emb)(idx3, table)
    return out.reshape(b, l, emb)


# speedup vs baseline: 9.1787x; 1.0010x over previous
import functools

import jax
import jax.numpy as jnp
from jax import lax
from jax.experimental import pallas as pl
from jax.experimental.pallas import tpu as pltpu
from jax.experimental.pallas import tpu_sc as plsc

_NC = 2
_NS = 16
_NW = _NC * _NS

_CH = 128


_NB = 5


def _make_sc_gather(tot, emb):
    per_w = tot // _NW
    n_chunks = per_w // _CH
    n_groups = n_chunks // _NB
    mesh = plsc.VectorSubcoreMesh(core_axis_name="c", subcore_axis_name="s")

    scratch = (
        [pltpu.VMEM((n_chunks, _CH), jnp.int32)]
        + [pltpu.VMEM((_CH, emb), jnp.float32) for _ in range(_NB)]
        + [pltpu.SemaphoreType.DMA for _ in range(2 * _NB)]
    )

    @functools.partial(
        pl.kernel,
        mesh=mesh,
        out_type=jax.ShapeDtypeStruct((tot, emb), jnp.float32),
        scratch_types=scratch,
    )
    def sc_gather(idx_hbm, table_hbm, out_hbm, idx_v, *bufs):
        rows = bufs[:_NB]
        gsem = bufs[_NB:2 * _NB]
        wsem = bufs[2 * _NB:]
        wid = lax.axis_index("s") * _NC + lax.axis_index("c")
        base = wid * per_w
        pltpu.sync_copy(idx_hbm.at[wid], idx_v)

        def gather(c, b):
            return pltpu.async_copy(table_hbm.at[idx_v.at[c]], rows[b], gsem[b])

        def write(c, b):
            return pltpu.async_copy(
                rows[b], out_hbm.at[pl.ds(base + c * _CH, _CH)], wsem[b])

        def wait_gather(c, b):
            pltpu.make_async_copy(
                table_hbm.at[idx_v.at[c]], rows[b], gsem[b]).wait()

        def wait_write(c, b):
            pltpu.make_async_copy(
                rows[b], out_hbm.at[pl.ds(base + c * _CH, _CH)], wsem[b]).wait()

        for b in range(_NB):
            gather(b, b)
        for b in range(_NB):
            wait_gather(b, b)
            write(b, b)

        def body(t, carry):
            c0 = t * _NB
            for b in range(_NB):
                wait_write(c0 + b - _NB, b)
                gather(c0 + b, b)
            for b in range(_NB):
                wait_gather(c0 + b, b)
                write(c0 + b, b)
            return carry

        lax.fori_loop(1, n_groups, body, 0)

        for b in range(_NB):
            wait_write((n_groups - 1) * _NB + b, b)

    return sc_gather


def kernel(inputs, table):
    b, l = inputs.shape
    vocab, emb = table.shape
    tot = b * l
    per_w = tot // _NW
    idx3 = inputs.reshape(_NW, per_w // _CH, _CH).astype(jnp.int32)
    out = _make_sc_gather(tot, emb)(idx3, table)
    return out.reshape(b, l, emb)
